# Initial kernel scaffold; baseline (speedup 1.0000x reference)
import jax
import jax.numpy as jnp
from jax.experimental import pallas as pl


def kernel(input, indices):
    raise NotImplementedError("write your pallas kernel here")


# speedup vs baseline: 4.3115x; 4.3115x over previous
import functools

import jax
import jax.numpy as jnp
from jax import lax
from jax.experimental import pallas as pl
from jax.experimental.pallas import tpu as pltpu
from jax.experimental.pallas import tpu_sc as plsc

R = 64
S = 50176
M = 401408
N_ELEM = R * S
N_OUT = R * M
PARTS = 256
PART = N_OUT // PARTS
NW = 32
PPW = PARTS // NW
CH = 8192
BASE_MAX = N_ELEM - CH - 16
INT_MAX = jnp.iinfo(jnp.int32).max


def _scatter_body(sk_hbm, sv_hbm, bnd_hbm, out_hbm, bnd_s, kbuf, vbuf, obuf):
    w = lax.axis_index("s") * 2 + lax.axis_index("c")
    pltpu.sync_copy(bnd_hbm, bnd_s)
    zeros = jnp.zeros((16,), jnp.float32)

    def part_body(p, carry):
        t = w * PPW + p
        klo = t * PART
        bv = bnd_s[pl.ds(t, 16)]
        lo = bv[0]
        hi = bv[1]

        def zbody(i, c):
            for u in range(8):
                obuf[pl.ds((i * 8 + u) * 16, 16)] = zeros
            return c

        lax.fori_loop(0, PART // 128, zbody, 0)

        lo_al = lax.bitwise_and(lo, jnp.int32(-8))
        nch = (hi - lo_al + CH - 1) // CH

        def cbody(c, cc):
            base = pl.multiple_of(
                jnp.minimum(lo_al + c * CH, jnp.int32(BASE_MAX)), 8
            )
            pltpu.sync_copy(sk_hbm.at[pl.ds(base, CH + 24)], kbuf)
            pltpu.sync_copy(sv_hbm.at[pl.ds(base, CH + 16)], vbuf)

            def scatter16(o):
                kv = kbuf[pl.ds(o, 16)]
                kn = kbuf[pl.ds(o + 1, 16)]
                vv = vbuf[pl.ds(o, 16)]
                m = (kv >= klo) & (kv < klo + PART) & (kv != kn)
                plsc.store_scatter(obuf, [kv - klo], vv, mask=m)

            def sbody(i, c2):
                for u in range(4):
                    scatter16((i * 4 + u) * 16)
                return c2

            lax.fori_loop(0, CH // 64, sbody, 0)
            scatter16(CH)
            return cc

        lax.fori_loop(0, nch, cbody, 0)

        pltpu.sync_copy(obuf, out_hbm.at[pl.ds(t * PART, PART)])
        return carry

    lax.fori_loop(0, PPW, part_body, 0)


_scatter = functools.partial(
    pl.kernel,
    mesh=plsc.VectorSubcoreMesh(core_axis_name="c", subcore_axis_name="s"),
    out_type=jax.ShapeDtypeStruct((N_OUT,), jnp.float32),
    compiler_params=pltpu.CompilerParams(needs_layout_passes=False),
    scratch_types=[
        pltpu.VMEM((272,), jnp.int32),
        pltpu.VMEM((CH + 24,), jnp.int32),
        pltpu.VMEM((CH + 16,), jnp.float32),
        pltpu.VMEM((PART,), jnp.float32),
    ],
)(_scatter_body)


def kernel(input, indices):
    N, C, D, H, W = input.shape
    flat_v = input.reshape(-1)
    flat_i = indices.astype(jnp.int32).reshape(R, S)
    rows = jnp.arange(R, dtype=jnp.int32)[:, None]
    gkey = (rows * M + flat_i).reshape(-1)
    sk, sv = lax.sort((gkey, flat_v), dimension=0, num_keys=1, is_stable=False)
    bnd = jnp.searchsorted(
        sk, jnp.arange(PARTS + 1, dtype=jnp.int32) * PART
    ).astype(jnp.int32)
    bnd = jnp.pad(bnd, (0, 15), constant_values=N_ELEM)
    sk_p = jnp.concatenate([sk, jnp.full((8,), INT_MAX, jnp.int32)])
    out = _scatter(sk_p, sv, bnd)
    return out.reshape(N, C, 2 * D, 2 * H, 2 * W)
